# baseline (speedup 1.0000x reference)
import jax
import jax.numpy as jnp
from jax import lax
from jax.experimental import pallas as pl
from jax.experimental.pallas import tpu as pltpu
from jax.experimental.pallas import tpu_sc as plsc

_H = 512
_W = 512
_NC = 2
_NS = 16
_NW = _NC * _NS
_L = 16

_CHUNK = 1024
_SUB = 128
_GROUPS = _CHUNK // _L
_ROWS_W = _H // _NW

_SC_PARAMS = pltpu.CompilerParams(
    needs_layout_passes=False, use_tc_tiling_on_sc=False)


def _floor_f32(x):
    xi = x.astype(jnp.int32)
    xf = xi.astype(jnp.float32)
    return jnp.where(xf > x, xf - 1.0, xf)


def _build_body(rgb_hbm, table_hbm, rgb_v, patch_v):
    wid = lax.axis_index("s") * _NC + lax.axis_index("c")
    lane = lax.iota(jnp.int32, _L)
    row0 = wid * _ROWS_W

    nmain = _ROWS_W * _W
    pitch = (_ROWS_W + 1) * _W
    wrap = ((row0 + _ROWS_W) & (_H - 1)) * _W
    for c in range(3):
        pltpu.sync_copy(rgb_hbm.at[pl.ds(c * _H * _W + row0 * _W, nmain)],
                        rgb_v.at[pl.ds(c * pitch, nmain)])
        pltpu.sync_copy(rgb_hbm.at[pl.ds(c * _H * _W + wrap, _W)],
                        rgb_v.at[pl.ds(c * pitch + nmain, _W)])

    for i in range(_ROWS_W):
        def grp(g, _):
            o = g * _L
            t = o + lane
            tn = (t + 1) & (_W - 1)
            for c in range(3):
                rs = c * pitch + i * _W
                p1 = rgb_v[pl.ds(rs + o, _L)]
                p2 = rgb_v[pl.ds(rs + _W + o, _L)]
                p3 = plsc.load_gather(rgb_v, [rs + tn])
                p4 = plsc.load_gather(rgb_v, [rs + _W + tn])
                plsc.store_scatter(patch_v, [t, jnp.full((_L,), c, jnp.int32)], p1)
                plsc.store_scatter(patch_v, [t, jnp.full((_L,), 3 + c, jnp.int32)], p2)
                plsc.store_scatter(patch_v, [t, jnp.full((_L,), 6 + c, jnp.int32)], p3)
                plsc.store_scatter(patch_v, [t, jnp.full((_L,), 9 + c, jnp.int32)], p4)
            return 0

        lax.fori_loop(0, _W // _L, grp, 0)
        pltpu.sync_copy(patch_v,
                        table_hbm.at[pl.ds((row0 + i) * _W, _W), :])


def _lookup_body(table_hbm, s_hbm, t_hbm, ox_hbm, oy_hbm, oz_hbm,
                 s_v, t_v, idx_v, ds_v, dt_v, patch_v, o_v,
                 sem_st, sem_g, sem_o):
    wid = lax.axis_index("s") * _NC + lax.axis_index("c")
    n = s_hbm.shape[0]
    npts_w = n // _NW
    nchunks = npts_w // _CHUNK
    base0 = wid * npts_w

    lane = lax.iota(jnp.int32, _L)
    out_hbms = (ox_hbm, oy_hbm, oz_hbm)

    def st_descs(k, par, clamp=False):
        base = base0 + k * _CHUNK
        if clamp:
            base = jnp.where(k < nchunks, base, base0)
        return (pltpu.make_async_copy(
                    s_hbm.at[pl.ds(base, _CHUNK)], s_v.at[par], sem_st),
                pltpu.make_async_copy(
                    t_hbm.at[pl.ds(base, _CHUNK)], t_v.at[par], sem_st))

    def fire_st(k, par):
        for d in st_descs(k, par, clamp=True):
            d.start()

    def wait_st(k, par):
        for d in st_descs(k, par, clamp=True):
            d.wait()

    def fire_g(par):
        pltpu.make_async_copy(
            table_hbm.at[idx_v.at[par]],
            patch_v.at[par],
            sem_g).start()

    def wait_g(par):
        pltpu.make_async_copy(
            table_hbm.at[pl.ds(0, _CHUNK), :], patch_v.at[par], sem_g).wait()

    def o_descs(k, par):
        base = base0 + k * _CHUNK
        return [pltpu.make_async_copy(
                    o_v.at[par, c], out_hbms[c].at[pl.ds(base, _CHUNK)], sem_o)
                for c in range(3)]

    def fire_o(k, par):
        for d in o_descs(k, par):
            d.start()

    def wait_o(k, par):
        for d in o_descs(k, par):
            d.wait()

    def pass_a(par):
        @plsc.parallel_loop(0, _GROUPS, unroll=4)
        def _(g):
            o = g * _L
            s = s_v[par, pl.ds(o, _L)] * jnp.float32(_H) + jnp.float32(_H - 0.5)
            t = t_v[par, pl.ds(o, _L)] * jnp.float32(_W) + jnp.float32(_W - 0.5)
            i0 = s.astype(jnp.int32)
            j0 = t.astype(jnp.int32)
            ridx = ((i0 & (_H - 1)) << 9) | (j0 & (_W - 1))
            idx_v[par, pl.ds(o, _L)] = ridx
            ds_v[par, pl.ds(o, _L)] = s - i0.astype(jnp.float32)
            dt_v[par, pl.ds(o, _L)] = t - j0.astype(jnp.float32)

    def pass_b(par):
        @plsc.parallel_loop(0, _GROUPS, unroll=4)
        def _(g):
            o = g * _L
            p = o + lane
            ds = ds_v[par, pl.ds(o, _L)]
            dt = dt_v[par, pl.ds(o, _L)]
            w4 = ds * dt
            w2 = dt - w4
            w3 = ds - w4
            w1 = (1.0 - ds) - w2
            for c in range(3):
                p1 = plsc.load_gather(
                    patch_v.at[par], [p, jnp.full((_L,), c, jnp.int32)])
                p2 = plsc.load_gather(
                    patch_v.at[par], [p, jnp.full((_L,), 3 + c, jnp.int32)])
                p3 = plsc.load_gather(
                    patch_v.at[par], [p, jnp.full((_L,), 6 + c, jnp.int32)])
                p4 = plsc.load_gather(
                    patch_v.at[par], [p, jnp.full((_L,), 9 + c, jnp.int32)])
                acc = w1 * p1 + w2 * p2 + w3 * p3 + w4 * p4
                o_v[par, c, pl.ds(o, _L)] = acc

    def step(k, r, do_wg, do_wo):
        wait_st(k, r)
        pass_a(r)
        fire_g(r)
        fire_st(k + 1, (r + 1) % 3)
        if do_wg:
            rb = (r + 1) % 3
            wait_g(rb)
            if do_wo:
                wait_o(k - 5, rb)
            pass_b(rb)
            fire_o(k - 2, rb)

    fire_st(0, 0)
    step(0, 0, do_wg=False, do_wo=False)
    step(1, 1, do_wg=False, do_wo=False)
    step(2, 2, do_wg=True, do_wo=False)
    step(3, 0, do_wg=True, do_wo=False)
    step(4, 1, do_wg=True, do_wo=False)

    def steady(m, _):
        k = 5 + m * 3
        step(k, 2, do_wg=True, do_wo=True)
        step(k + 1, 0, do_wg=True, do_wo=True)
        step(k + 2, 1, do_wg=True, do_wo=True)
        return 0

    lax.fori_loop(0, (nchunks - 5) // 3, steady, 0)

    last = nchunks - 1
    for k in (last - 1, last):
        rb = k % 3
        wait_g(rb)
        wait_o(k - 3, rb)
        pass_b(rb)
        fire_o(k, rb)
    for k in (last - 2, last - 1, last):
        wait_o(k, k % 3)
    wait_st(nchunks, nchunks % 3)


def kernel(img, st):
    n = st.shape[0]
    assert n % (_NW * _CHUNK) == 0
    assert (n // (_NW * _CHUNK) - 5) % 3 == 0
    rgb1d = img.transpose(2, 0, 1).reshape(3 * _H * _W)
    s1d = st[:, 0]
    t1d = st[:, 1]

    mesh = plsc.VectorSubcoreMesh(core_axis_name="c", subcore_axis_name="s")

    build = pl.kernel(
        _build_body,
        out_type=jax.ShapeDtypeStruct((_H * _W, 16), jnp.float32),
        mesh=mesh,
        compiler_params=_SC_PARAMS,
        scratch_types=[
            pltpu.VMEM((3 * (_ROWS_W + 1) * _W,), jnp.float32),
            pltpu.VMEM((_W, 16), jnp.float32),
        ],
    )
    table = build(rgb1d)

    lookup = pl.kernel(
        _lookup_body,
        out_type=[jax.ShapeDtypeStruct((n,), jnp.float32)] * 3,
        mesh=mesh,
        compiler_params=_SC_PARAMS,
        scratch_types=[
            pltpu.VMEM((3, _CHUNK), jnp.float32),
            pltpu.VMEM((3, _CHUNK), jnp.float32),
            pltpu.VMEM((3, _CHUNK), jnp.int32),
            pltpu.VMEM((3, _CHUNK), jnp.float32),
            pltpu.VMEM((3, _CHUNK), jnp.float32),
            pltpu.VMEM((3, _CHUNK, 16), jnp.float32),
            pltpu.VMEM((3, 3, _CHUNK), jnp.float32),
            pltpu.SemaphoreType.DMA,
            pltpu.SemaphoreType.DMA,
            pltpu.SemaphoreType.DMA,
        ],
    )
    ox, oy, oz = lookup(table, s1d, t1d)
    return jnp.stack([ox, oy, oz], axis=1)
